# baseline (speedup 1.0000x reference)
import dataclasses
import functools

import jax
import jax.numpy as jnp
from jax import lax
from jax.experimental import pallas as pl
from jax.experimental.pallas import tpu as pltpu
from jax.experimental.pallas import tpu_sc as plsc

DIM = 64
BATCH = 16384
PAD_DIM = 128
NUM_REL = 100000
TAB_ROWS = 51200
Q_ROWS = BATCH // 2
NUM_CORES = 2
NUM_SUBCORES = 16
NUM_WORKERS = NUM_CORES * NUM_SUBCORES
ROWS_PER_WORKER = BATCH // NUM_WORKERS
STAGE_ROWS = 128
NUM_STAGES = ROWS_PER_WORKER // STAGE_ROWS
LANES = 16
DIM_CHUNKS = DIM // LANES
GROUPS_PER_STAGE = STAGE_ROWS // LANES

TR_COLS = 2048
TR_GRID = TAB_ROWS // TR_COLS
TR_OFF = TAB_ROWS // TR_COLS
TR_LAST = (NUM_REL - 1) // TR_COLS


def _pack2_kernel(a_ref, b_ref, out_ref):
  out_ref[...] = jnp.concatenate([a_ref[...].T, b_ref[...].T], axis=1)


def _pack_diag(dt):
  return pl.pallas_call(
      _pack2_kernel,
      grid=(TR_GRID,),
      in_specs=[
          pl.BlockSpec((DIM, TR_COLS), lambda i: (0, i)),
          pl.BlockSpec((DIM, TR_COLS),
                       lambda i: (0, jnp.minimum(i + TR_OFF, TR_LAST))),
      ],
      out_specs=pl.BlockSpec((TR_COLS, PAD_DIM), lambda i: (i, 0)),
      out_shape=jax.ShapeDtypeStruct((TAB_ROWS, PAD_DIM), jnp.float32),
  )(dt, dt)


QC_COLS = 4096


def _packq_kernel(ha_ref, ta_ref, hb_ref, tb_ref, out_ref):
  a = ha_ref[...] * ta_ref[...]
  b = hb_ref[...] * tb_ref[...]
  out_ref[...] = jnp.concatenate([a.T, b.T], axis=1)


def _pack_q(ht, tt):
  qoff = Q_ROWS // QC_COLS
  return pl.pallas_call(
      _packq_kernel,
      grid=(Q_ROWS // QC_COLS,),
      in_specs=[
          pl.BlockSpec((DIM, QC_COLS), lambda i: (0, i)),
          pl.BlockSpec((DIM, QC_COLS), lambda i: (0, i)),
          pl.BlockSpec((DIM, QC_COLS), lambda i: (0, i + 2)),
          pl.BlockSpec((DIM, QC_COLS), lambda i: (0, i + 2)),
      ],
      out_specs=pl.BlockSpec((QC_COLS, PAD_DIM), lambda i: (i, 0)),
      out_shape=jax.ShapeDtypeStruct((Q_ROWS, PAD_DIM), jnp.float32),
  )(ht, tt, ht, tt)


def _sc_kernel(diag2_hbm, idx_hbm, q_hbm, out_hbm,
               idx_v, row_v, rel0, rel1, q_v, out_v,
               sem_q, sem_g0, sem_g1):
  wid = lax.axis_index("s") * NUM_CORES + lax.axis_index("c")
  base = wid * ROWS_PER_WORKER
  qhalf = wid >= (NUM_WORKERS // 2)
  qbase = base - jnp.where(qhalf, Q_ROWS, 0)
  qoff = jnp.where(qhalf, DIM, 0)

  pltpu.sync_copy(idx_hbm.at[wid], idx_v)
  copy_q = pltpu.async_copy(
      q_hbm.at[pl.ds(qbase, ROWS_PER_WORKER)], q_v, sem_q)

  for j in range(NUM_STAGES):
    for p in range(GROUPS_PER_STAGE):
      sl = pl.ds(p * LANES, LANES)
      iv = idx_v[j, sl]
      row_v[j, sl] = iv - jnp.where(iv >= TAB_ROWS, TAB_ROWS, 0)

  rel = (rel0, rel1)
  sems = (sem_g0, sem_g1)
  lane = lax.iota(jnp.int32, LANES)

  def compute_stage(s, relbuf):
    @pl.loop(0, GROUPS_PER_STAGE)
    def _(g):
      hi = idx_v[s, pl.ds(g * LANES, LANES)] >= TAB_ROWS
      res = [jnp.zeros((LANES,), jnp.float32) for _ in range(4)]
      for k in range(LANES):
        li = g * LANES + k
        qrow = s * STAGE_ROWS + g * LANES + k
        s0 = None
        s1 = None
        for c in range(DIM_CHUNKS):
          qc = q_v[qrow, pl.ds(qoff + c * LANES, LANES)]
          a0 = qc * relbuf[li, pl.ds(c * LANES, LANES)]
          a1 = qc * relbuf[li, pl.ds(DIM + c * LANES, LANES)]
          s0 = a0 if s0 is None else s0 + a0
          s1 = a1 if s1 is None else s1 + a1
        sel = lane == k
        ch = k & 1
        res[ch] = jnp.where(sel, jnp.sum(s0), res[ch])
        res[2 + ch] = jnp.where(sel, jnp.sum(s1), res[2 + ch])
      lo = res[0] + res[1]
      hi_sum = res[2] + res[3]
      out_v[pl.ds(s * STAGE_ROWS + g * LANES, LANES)] = (
          jnp.where(hi, hi_sum, lo))

  gathers = [None] * NUM_STAGES
  gathers[0] = pltpu.async_copy(diag2_hbm.at[row_v.at[0]], rel[0], sems[0])
  copy_q.wait()
  for s in range(NUM_STAGES):
    if s + 1 < NUM_STAGES:
      gathers[s + 1] = pltpu.async_copy(
          diag2_hbm.at[row_v.at[s + 1]], rel[(s + 1) % 2], sems[(s + 1) % 2])
    gathers[s].wait()
    compute_stage(s, rel[s % 2])

  pltpu.sync_copy(out_v, out_hbm.at[pl.ds(base, ROWS_PER_WORKER)])


@jax.jit
def _dist_mult(h, r, t, diag):
  idx = r.astype(jnp.int32).reshape(NUM_WORKERS, NUM_STAGES, STAGE_ROWS)
  diag2 = _pack_diag(diag.T)
  q = _pack_q(h.T, t.T)
  mesh = plsc.VectorSubcoreMesh(core_axis_name="c", subcore_axis_name="s")
  cp = pltpu.CompilerParams()
  for field, value in (("needs_layout_passes", False),
                       ("use_tc_tiling_on_sc", True)):
    if field in pltpu.CompilerParams.__dataclass_fields__:
      cp = dataclasses.replace(cp, **{field: value})
  run = pl.kernel(
      _sc_kernel,
      out_type=jax.ShapeDtypeStruct((BATCH,), jnp.float32),
      mesh=mesh,
      compiler_params=cp,
      scratch_types=[
          pltpu.VMEM((NUM_STAGES, STAGE_ROWS), jnp.int32),
          pltpu.VMEM((NUM_STAGES, STAGE_ROWS), jnp.int32),
          pltpu.VMEM((STAGE_ROWS, PAD_DIM), jnp.float32),
          pltpu.VMEM((STAGE_ROWS, PAD_DIM), jnp.float32),
          pltpu.VMEM((ROWS_PER_WORKER, PAD_DIM), jnp.float32),
          pltpu.VMEM((ROWS_PER_WORKER,), jnp.float32),
          pltpu.SemaphoreType.DMA,
          pltpu.SemaphoreType.DMA,
          pltpu.SemaphoreType.DMA,
      ],
  )
  return run(diag2, idx, q)


def kernel(h, r, t, diag):
  return _dist_mult(h, r, t, diag)


# speedup vs baseline: 1.1254x; 1.1254x over previous
import dataclasses
import functools

import jax
import jax.numpy as jnp
from jax import lax
from jax.experimental import pallas as pl
from jax.experimental.pallas import tpu as pltpu
from jax.experimental.pallas import tpu_sc as plsc

DIM = 64
BATCH = 16384
PAD_DIM = 128
NUM_REL = 100000
TAB_ROWS = 51200
Q_ROWS = BATCH // 2
NUM_CORES = 2
NUM_SUBCORES = 16
NUM_WORKERS = NUM_CORES * NUM_SUBCORES
ROWS_PER_WORKER = BATCH // NUM_WORKERS
STAGE_ROWS = 128
NUM_STAGES = ROWS_PER_WORKER // STAGE_ROWS
LANES = 16
DIM_CHUNKS = DIM // LANES
GROUPS_PER_STAGE = STAGE_ROWS // LANES

TR_COLS = 6400
TR_GRID = TAB_ROWS // TR_COLS
TR_OFF = TAB_ROWS // TR_COLS
TR_LAST = (NUM_REL - 1) // TR_COLS


def _pack2_kernel(a_ref, b_ref, out_ref):
  out_ref[...] = jnp.concatenate([a_ref[...].T, b_ref[...].T], axis=1)


def _pack_diag(dt):
  return pl.pallas_call(
      _pack2_kernel,
      grid=(TR_GRID,),
      in_specs=[
          pl.BlockSpec((DIM, TR_COLS), lambda i: (0, i)),
          pl.BlockSpec((DIM, TR_COLS),
                       lambda i: (0, jnp.minimum(i + TR_OFF, TR_LAST))),
      ],
      out_specs=pl.BlockSpec((TR_COLS, PAD_DIM), lambda i: (i, 0)),
      out_shape=jax.ShapeDtypeStruct((TAB_ROWS, PAD_DIM), jnp.float32),
  )(dt, dt)


QC_COLS = 4096


def _packq_kernel(ha_ref, ta_ref, hb_ref, tb_ref, out_ref):
  a = ha_ref[...] * ta_ref[...]
  b = hb_ref[...] * tb_ref[...]
  out_ref[...] = jnp.concatenate([a.T, b.T], axis=1)


def _pack_q(ht, tt):
  qoff = Q_ROWS // QC_COLS
  return pl.pallas_call(
      _packq_kernel,
      grid=(Q_ROWS // QC_COLS,),
      in_specs=[
          pl.BlockSpec((DIM, QC_COLS), lambda i: (0, i)),
          pl.BlockSpec((DIM, QC_COLS), lambda i: (0, i)),
          pl.BlockSpec((DIM, QC_COLS), lambda i: (0, i + 2)),
          pl.BlockSpec((DIM, QC_COLS), lambda i: (0, i + 2)),
      ],
      out_specs=pl.BlockSpec((QC_COLS, PAD_DIM), lambda i: (i, 0)),
      out_shape=jax.ShapeDtypeStruct((Q_ROWS, PAD_DIM), jnp.float32),
  )(ht, tt, ht, tt)


def _sc_kernel(diag2_hbm, idx_hbm, q_hbm, out_hbm,
               idx_v, row_v, rel0, rel1, q_v, out_v,
               sem_q, sem_g0, sem_g1):
  wid = lax.axis_index("s") * NUM_CORES + lax.axis_index("c")
  base = wid * ROWS_PER_WORKER
  qhalf = wid >= (NUM_WORKERS // 2)
  qbase = base - jnp.where(qhalf, Q_ROWS, 0)
  qoff = jnp.where(qhalf, DIM, 0)

  pltpu.sync_copy(idx_hbm.at[wid], idx_v)
  copy_q = pltpu.async_copy(
      q_hbm.at[pl.ds(qbase, ROWS_PER_WORKER)], q_v, sem_q)

  for j in range(NUM_STAGES):
    for p in range(GROUPS_PER_STAGE):
      sl = pl.ds(p * LANES, LANES)
      iv = idx_v[j, sl]
      row_v[j, sl] = iv - jnp.where(iv >= TAB_ROWS, TAB_ROWS, 0)

  rel = (rel0, rel1)
  sems = (sem_g0, sem_g1)
  lane = lax.iota(jnp.int32, LANES)

  def compute_stage(s, relbuf):
    @pl.loop(0, GROUPS_PER_STAGE)
    def _(g):
      hi = idx_v[s, pl.ds(g * LANES, LANES)] >= TAB_ROWS
      res = [jnp.zeros((LANES,), jnp.float32) for _ in range(4)]
      for k in range(LANES):
        li = g * LANES + k
        qrow = s * STAGE_ROWS + g * LANES + k
        s0 = None
        s1 = None
        for c in range(DIM_CHUNKS):
          qc = q_v[qrow, pl.ds(qoff + c * LANES, LANES)]
          a0 = qc * relbuf[li, pl.ds(c * LANES, LANES)]
          a1 = qc * relbuf[li, pl.ds(DIM + c * LANES, LANES)]
          s0 = a0 if s0 is None else s0 + a0
          s1 = a1 if s1 is None else s1 + a1
        sel = lane == k
        ch = k & 1
        res[ch] = jnp.where(sel, jnp.sum(s0), res[ch])
        res[2 + ch] = jnp.where(sel, jnp.sum(s1), res[2 + ch])
      lo = res[0] + res[1]
      hi_sum = res[2] + res[3]
      out_v[pl.ds(s * STAGE_ROWS + g * LANES, LANES)] = (
          jnp.where(hi, hi_sum, lo))

  gathers = [None] * NUM_STAGES
  gathers[0] = pltpu.async_copy(diag2_hbm.at[row_v.at[0]], rel[0], sems[0])
  copy_q.wait()
  for s in range(NUM_STAGES):
    if s + 1 < NUM_STAGES:
      gathers[s + 1] = pltpu.async_copy(
          diag2_hbm.at[row_v.at[s + 1]], rel[(s + 1) % 2], sems[(s + 1) % 2])
    gathers[s].wait()
    compute_stage(s, rel[s % 2])

  pltpu.sync_copy(out_v, out_hbm.at[pl.ds(base, ROWS_PER_WORKER)])


@jax.jit
def _dist_mult(h, r, t, diag):
  idx = r.astype(jnp.int32).reshape(NUM_WORKERS, NUM_STAGES, STAGE_ROWS)
  diag2 = _pack_diag(diag.T)
  q = _pack_q(h.T, t.T)
  mesh = plsc.VectorSubcoreMesh(core_axis_name="c", subcore_axis_name="s")
  cp = pltpu.CompilerParams()
  for field, value in (("needs_layout_passes", False),
                       ("use_tc_tiling_on_sc", True)):
    if field in pltpu.CompilerParams.__dataclass_fields__:
      cp = dataclasses.replace(cp, **{field: value})
  run = pl.kernel(
      _sc_kernel,
      out_type=jax.ShapeDtypeStruct((BATCH,), jnp.float32),
      mesh=mesh,
      compiler_params=cp,
      scratch_types=[
          pltpu.VMEM((NUM_STAGES, STAGE_ROWS), jnp.int32),
          pltpu.VMEM((NUM_STAGES, STAGE_ROWS), jnp.int32),
          pltpu.VMEM((STAGE_ROWS, PAD_DIM), jnp.float32),
          pltpu.VMEM((STAGE_ROWS, PAD_DIM), jnp.float32),
          pltpu.VMEM((ROWS_PER_WORKER, PAD_DIM), jnp.float32),
          pltpu.VMEM((ROWS_PER_WORKER,), jnp.float32),
          pltpu.SemaphoreType.DMA,
          pltpu.SemaphoreType.DMA,
          pltpu.SemaphoreType.DMA,
      ],
  )
  return run(diag2, idx, q)


def kernel(h, r, t, diag):
  return _dist_mult(h, r, t, diag)


# speedup vs baseline: 1.1883x; 1.0559x over previous
import dataclasses
import functools

import jax
import jax.numpy as jnp
from jax import lax
from jax.experimental import pallas as pl
from jax.experimental.pallas import tpu as pltpu
from jax.experimental.pallas import tpu_sc as plsc

DIM = 64
BATCH = 16384
PAD_DIM = 128
NUM_REL = 100000
NUM_CORES = 2
NUM_SUBCORES = 16
NUM_WORKERS = NUM_CORES * NUM_SUBCORES
ROWS_PER_WORKER = BATCH // NUM_WORKERS
STAGE_ROWS = 128
NUM_STAGES = ROWS_PER_WORKER // STAGE_ROWS
LANES = 16
DIM_CHUNKS = DIM // LANES
GROUPS_PER_STAGE = STAGE_ROWS // LANES

TR_COLS = 16384


def _pack_diag_kernel(dt_ref, out_ref):
  x = dt_ref[...].T
  out_ref[...] = jnp.concatenate(
      [x, jnp.zeros((TR_COLS, PAD_DIM - DIM), jnp.float32)], axis=1)


def _pack_diag(dt):
  return pl.pallas_call(
      _pack_diag_kernel,
      grid=(-(-NUM_REL // TR_COLS),),
      in_specs=[pl.BlockSpec((DIM, TR_COLS), lambda i: (0, i))],
      out_specs=pl.BlockSpec((TR_COLS, PAD_DIM), lambda i: (i, 0)),
      out_shape=jax.ShapeDtypeStruct((NUM_REL, PAD_DIM), jnp.float32),
  )(dt)


def _pack_q_kernel(ht_ref, tt_ref, out_ref):
  x = (ht_ref[...] * tt_ref[...]).T
  out_ref[...] = jnp.concatenate(
      [x, jnp.zeros((TR_COLS, PAD_DIM - DIM), jnp.float32)], axis=1)


def _pack_q(ht, tt):
  return pl.pallas_call(
      _pack_q_kernel,
      grid=(BATCH // TR_COLS,),
      in_specs=[pl.BlockSpec((DIM, TR_COLS), lambda i: (0, i)),
                pl.BlockSpec((DIM, TR_COLS), lambda i: (0, i))],
      out_specs=pl.BlockSpec((TR_COLS, PAD_DIM), lambda i: (i, 0)),
      out_shape=jax.ShapeDtypeStruct((BATCH, PAD_DIM), jnp.float32),
  )(ht, tt)


def _sc_kernel(diagp_hbm, idx_hbm, q_hbm, out_hbm,
               idx_v, rel0, rel1, q_v, out_v,
               sem_q, sem_g0, sem_g1):
  wid = lax.axis_index("s") * NUM_CORES + lax.axis_index("c")
  base = wid * ROWS_PER_WORKER

  pltpu.sync_copy(idx_hbm.at[wid], idx_v)
  copy_q = pltpu.async_copy(
      q_hbm.at[pl.ds(base, ROWS_PER_WORKER)], q_v, sem_q)

  rel = (rel0, rel1)
  sems = (sem_g0, sem_g1)
  lane = lax.iota(jnp.int32, LANES)

  def compute_stage(s, relbuf):
    @pl.loop(0, GROUPS_PER_STAGE)
    def _(g):
      res = [jnp.zeros((LANES,), jnp.float32) for _ in range(2)]
      for k in range(LANES):
        li = g * LANES + k
        qrow = s * STAGE_ROWS + g * LANES + k
        acc = None
        for c in range(DIM_CHUNKS):
          term = (q_v[qrow, pl.ds(c * LANES, LANES)]
                  * relbuf[li, pl.ds(c * LANES, LANES)])
          acc = term if acc is None else acc + term
        ch = k & 1
        res[ch] = jnp.where(lane == k, jnp.sum(acc), res[ch])
      out_v[pl.ds(s * STAGE_ROWS + g * LANES, LANES)] = res[0] + res[1]

  gathers = [None] * NUM_STAGES
  gathers[0] = pltpu.async_copy(diagp_hbm.at[idx_v.at[0]], rel[0], sems[0])
  copy_q.wait()
  for s in range(NUM_STAGES):
    if s + 1 < NUM_STAGES:
      gathers[s + 1] = pltpu.async_copy(
          diagp_hbm.at[idx_v.at[s + 1]], rel[(s + 1) % 2], sems[(s + 1) % 2])
    gathers[s].wait()
    compute_stage(s, rel[s % 2])

  pltpu.sync_copy(out_v, out_hbm.at[pl.ds(base, ROWS_PER_WORKER)])


@jax.jit
def _dist_mult(h, r, t, diag):
  idx = r.astype(jnp.int32).reshape(NUM_WORKERS, NUM_STAGES, STAGE_ROWS)
  diagp = _pack_diag(diag.T)
  q = _pack_q(h.T, t.T)
  mesh = plsc.VectorSubcoreMesh(core_axis_name="c", subcore_axis_name="s")
  cp = pltpu.CompilerParams()
  for field, value in (("needs_layout_passes", False),
                       ("use_tc_tiling_on_sc", True)):
    if field in pltpu.CompilerParams.__dataclass_fields__:
      cp = dataclasses.replace(cp, **{field: value})
  run = pl.kernel(
      _sc_kernel,
      out_type=jax.ShapeDtypeStruct((BATCH,), jnp.float32),
      mesh=mesh,
      compiler_params=cp,
      scratch_types=[
          pltpu.VMEM((NUM_STAGES, STAGE_ROWS), jnp.int32),
          pltpu.VMEM((STAGE_ROWS, PAD_DIM), jnp.float32),
          pltpu.VMEM((STAGE_ROWS, PAD_DIM), jnp.float32),
          pltpu.VMEM((ROWS_PER_WORKER, PAD_DIM), jnp.float32),
          pltpu.VMEM((ROWS_PER_WORKER,), jnp.float32),
          pltpu.SemaphoreType.DMA,
          pltpu.SemaphoreType.DMA,
          pltpu.SemaphoreType.DMA,
      ],
  )
  return run(diagp, idx, q)


def kernel(h, r, t, diag):
  return _dist_mult(h, r, t, diag)


# speedup vs baseline: 1.2173x; 1.0244x over previous
import dataclasses
import functools

import jax
import jax.numpy as jnp
from jax import lax
from jax.experimental import pallas as pl
from jax.experimental.pallas import tpu as pltpu
from jax.experimental.pallas import tpu_sc as plsc

DIM = 64
BATCH = 16384
PAD_DIM = 128
NUM_REL = 100000
NUM_CORES = 2
NUM_SUBCORES = 16
NUM_WORKERS = NUM_CORES * NUM_SUBCORES
ROWS_PER_WORKER = BATCH // NUM_WORKERS
STAGE_ROWS = 128
NUM_STAGES = ROWS_PER_WORKER // STAGE_ROWS
LANES = 16
DIM_CHUNKS = DIM // LANES
GROUPS_PER_STAGE = STAGE_ROWS // LANES

TR_COLS = 16384


def _pack_diag_kernel(dt_ref, out_ref):
  x = dt_ref[...].T
  out_ref[...] = jnp.concatenate(
      [x, jnp.zeros((TR_COLS, PAD_DIM - DIM), jnp.float32)], axis=1)


def _pack_diag(dt):
  return pl.pallas_call(
      _pack_diag_kernel,
      grid=(-(-NUM_REL // TR_COLS),),
      in_specs=[pl.BlockSpec((DIM, TR_COLS), lambda i: (0, i))],
      out_specs=pl.BlockSpec((TR_COLS, PAD_DIM), lambda i: (i, 0)),
      out_shape=jax.ShapeDtypeStruct((NUM_REL, PAD_DIM), jnp.float32),
  )(dt)


QC_COLS = 4096
Q_ROWS = BATCH // 2


def _pack_q_kernel(ha_ref, ta_ref, hb_ref, tb_ref, out_ref):
  a = ha_ref[...] * ta_ref[...]
  b = hb_ref[...] * tb_ref[...]
  out_ref[...] = jnp.concatenate([a.T, b.T], axis=1)


def _pack_q(ht, tt):
  return pl.pallas_call(
      _pack_q_kernel,
      grid=(Q_ROWS // QC_COLS,),
      in_specs=[
          pl.BlockSpec((DIM, QC_COLS), lambda i: (0, i)),
          pl.BlockSpec((DIM, QC_COLS), lambda i: (0, i)),
          pl.BlockSpec((DIM, QC_COLS), lambda i: (0, i + Q_ROWS // QC_COLS)),
          pl.BlockSpec((DIM, QC_COLS), lambda i: (0, i + Q_ROWS // QC_COLS)),
      ],
      out_specs=pl.BlockSpec((QC_COLS, PAD_DIM), lambda i: (i, 0)),
      out_shape=jax.ShapeDtypeStruct((Q_ROWS, PAD_DIM), jnp.float32),
  )(ht, tt, ht, tt)


def _sc_kernel(diagp_hbm, idx_hbm, q_hbm, out_hbm,
               idx_v, rel0, rel1, q_v, out_v,
               sem_q, sem_g0, sem_g1):
  wid = lax.axis_index("s") * NUM_CORES + lax.axis_index("c")
  half = ROWS_PER_WORKER // 2
  base = wid * half

  pltpu.sync_copy(idx_hbm.at[wid], idx_v)
  copy_q = pltpu.async_copy(q_hbm.at[pl.ds(base, half)], q_v, sem_q)

  rel = (rel0, rel1)
  sems = (sem_g0, sem_g1)
  lane = lax.iota(jnp.int32, LANES)

  def compute_stage(s, relbuf):
    qoff = (s // 2) * DIM

    @pl.loop(0, GROUPS_PER_STAGE)
    def _(g):
      res = [jnp.zeros((LANES,), jnp.float32) for _ in range(2)]
      for k in range(LANES):
        li = g * LANES + k
        qrow = (s % 2) * STAGE_ROWS + g * LANES + k
        acc = None
        for c in range(DIM_CHUNKS):
          term = (q_v[qrow, pl.ds(qoff + c * LANES, LANES)]
                  * relbuf[li, pl.ds(c * LANES, LANES)])
          acc = term if acc is None else acc + term
        ch = k & 1
        res[ch] = jnp.where(lane == k, jnp.sum(acc), res[ch])
      out_v[pl.ds(s * STAGE_ROWS + g * LANES, LANES)] = res[0] + res[1]

  gathers = [None] * NUM_STAGES
  gathers[0] = pltpu.async_copy(diagp_hbm.at[idx_v.at[0]], rel[0], sems[0])
  copy_q.wait()
  for s in range(NUM_STAGES):
    if s + 1 < NUM_STAGES:
      gathers[s + 1] = pltpu.async_copy(
          diagp_hbm.at[idx_v.at[s + 1]], rel[(s + 1) % 2], sems[(s + 1) % 2])
    gathers[s].wait()
    compute_stage(s, rel[s % 2])

  pltpu.sync_copy(out_v.at[pl.ds(0, half)], out_hbm.at[pl.ds(base, half)])
  pltpu.sync_copy(out_v.at[pl.ds(half, half)],
                  out_hbm.at[pl.ds(Q_ROWS + base, half)])


@jax.jit
def _dist_mult(h, r, t, diag):
  idx = (r.astype(jnp.int32)
         .reshape(2, NUM_WORKERS, ROWS_PER_WORKER // 2)
         .transpose(1, 0, 2)
         .reshape(NUM_WORKERS, NUM_STAGES, STAGE_ROWS))
  diagp = _pack_diag(diag.T)
  q = _pack_q(h.T, t.T)
  mesh = plsc.VectorSubcoreMesh(core_axis_name="c", subcore_axis_name="s")
  cp = pltpu.CompilerParams()
  for field, value in (("needs_layout_passes", False),
                       ("use_tc_tiling_on_sc", True)):
    if field in pltpu.CompilerParams.__dataclass_fields__:
      cp = dataclasses.replace(cp, **{field: value})
  run = pl.kernel(
      _sc_kernel,
      out_type=jax.ShapeDtypeStruct((BATCH,), jnp.float32),
      mesh=mesh,
      compiler_params=cp,
      scratch_types=[
          pltpu.VMEM((NUM_STAGES, STAGE_ROWS), jnp.int32),
          pltpu.VMEM((STAGE_ROWS, PAD_DIM), jnp.float32),
          pltpu.VMEM((STAGE_ROWS, PAD_DIM), jnp.float32),
          pltpu.VMEM((ROWS_PER_WORKER // 2, PAD_DIM), jnp.float32),
          pltpu.VMEM((ROWS_PER_WORKER,), jnp.float32),
          pltpu.SemaphoreType.DMA,
          pltpu.SemaphoreType.DMA,
          pltpu.SemaphoreType.DMA,
      ],
  )
  return run(diagp, idx, q)


def kernel(h, r, t, diag):
  return _dist_mult(h, r, t, diag)
